# speedup vs baseline: 1.8820x; 1.8820x over previous
import jax
import jax.numpy as jnp
from jax.experimental import pallas as pl
from jax.experimental.pallas import tpu as pltpu

IN_DIM = 1024
OUT_DIM = 1024
K = 2048
B_TOK = 8192
TM = 512


def _fused_body(x_ref, a_ref, b_ref, out_ref, inner_ref,
                inv_norm_ref, a_bf_ref, b_bf_ref):
    step = pl.program_id(0)

    @pl.when(step == 0)
    def _prep():
        a32 = a_ref[...]
        inv_norm_ref[...] = jax.lax.rsqrt(
            jnp.sum(a32 * a32, axis=0, keepdims=True))
        a_bf_ref[...] = a32.astype(jnp.bfloat16)
        b_bf_ref[...] = b_ref[...].astype(jnp.bfloat16)

    x_bf = x_ref[...].astype(jnp.bfloat16)
    inner = jnp.dot(x_bf, a_bf_ref[...],
                    preferred_element_type=jnp.float32)
    inner = inner * inv_norm_ref[...]
    inner_ref[...] = inner
    out_ref[...] = jnp.dot(inner.astype(jnp.bfloat16), b_bf_ref[...],
                           preferred_element_type=jnp.float32)


def kernel(x, A, Bm):
    n_tiles = B_TOK // TM
    out, inner = pl.pallas_call(
        _fused_body,
        grid=(n_tiles,),
        in_specs=[
            pl.BlockSpec((TM, IN_DIM), lambda i: (i, 0)),
            pl.BlockSpec((IN_DIM, K), lambda i: (0, 0)),
            pl.BlockSpec((K, OUT_DIM), lambda i: (0, 0)),
        ],
        out_specs=[
            pl.BlockSpec((TM, OUT_DIM), lambda i: (i, 0)),
            pl.BlockSpec((TM, K), lambda i: (i, 0)),
        ],
        out_shape=[
            jax.ShapeDtypeStruct((B_TOK, OUT_DIM), jnp.float32),
            jax.ShapeDtypeStruct((B_TOK, K), jnp.float32),
        ],
        scratch_shapes=[
            pltpu.VMEM((1, K), jnp.float32),
            pltpu.VMEM((IN_DIM, K), jnp.bfloat16),
            pltpu.VMEM((K, OUT_DIM), jnp.bfloat16),
        ],
    )(x, A, Bm)
    return (out, inner)
